# baseline (speedup 1.0000x reference)
import functools

import jax
import jax.numpy as jnp
from jax import lax
from jax.experimental import pallas as pl
from jax.experimental.pallas import tpu as pltpu
from jax.experimental.pallas import tpu_sc as plsc

_INFO = plsc.get_sparse_core_info()
_NC, _NS, _L = _INFO.num_cores, _INFO.num_subcores, _INFO.num_lanes
_NW = _NC * _NS


def _sc_body(n, c, d, rw, ch, nchunk, x_hbm, emb_hbm, out_hbm,
             x_a, x_b, idx_v, rows_v, sem_a, sem_b, sem_g):
    wid = lax.axis_index("s") * _NC + lax.axis_index("c")
    row0 = wid * rw
    lanes = lax.iota(jnp.int32, _L)
    neg_inf = jnp.full((_L,), -jnp.inf, jnp.float32)
    zeros_i = jnp.zeros((_L,), jnp.int32)

    bufs = (x_a, x_b)
    sems = (sem_a, sem_b)

    def start_load(chunk, buf):
        return pltpu.make_async_copy(
            x_hbm.at[pl.ds(row0 + chunk * ch, ch)], bufs[buf], sems[buf])

    start_load(0, 0).start()

    ngrp = ch // _L
    nstr = 4

    for chunk in range(nchunk):
        buf = chunk % 2
        if chunk + 1 < nchunk:
            start_load(chunk + 1, 1 - buf).start()
        start_load(chunk, buf).wait()

        rows16 = [lanes + g * _L for g in range(ngrp)]
        cms = [neg_inf] * (ngrp * nstr)
        cis = [zeros_i] * (ngrp * nstr)
        jvs = [jnp.full((_L,), s, jnp.int32) for s in range(nstr)]
        step = jnp.full((_L,), nstr, jnp.int32)

        @plsc.parallel_loop(0, c // nstr, unroll=2,
                            carry=(tuple(cms), tuple(cis), tuple(jvs)))
        def jloop(t, carry, buf=buf, rows16=rows16):
            cms, cis, jvs = (list(x) for x in carry)
            for s in range(nstr):
                for g in range(ngrp):
                    k = g * nstr + s
                    v = plsc.load_gather(bufs[buf], [rows16[g], jvs[s]])
                    upd = v > cms[k]
                    cms[k] = jnp.where(upd, v, cms[k])
                    cis[k] = jnp.where(upd, jvs[s], cis[k])
            for s in range(nstr):
                jvs[s] = jvs[s] + step
            return tuple(cms), tuple(cis), tuple(jvs)

        cms, cis, _ = jloop
        for g in range(ngrp):
            cm, ci = cms[g * nstr], cis[g * nstr]
            for s in range(1, nstr):
                cmb, cib = cms[g * nstr + s], cis[g * nstr + s]
                upd = (cmb > cm) | ((cmb == cm) & (cib < ci))
                cm = jnp.where(upd, cmb, cm)
                ci = jnp.where(upd, cib, ci)
            idx_v[pl.ds(chunk * ch + g * _L, _L)] = ci

    gchunk = 128
    for q in range(rw // gchunk):
        cp = pltpu.make_async_copy(
            emb_hbm.at[idx_v.at[pl.ds(q * gchunk, gchunk)]], rows_v, sem_g)
        cp.start()
        cp.wait()
        pltpu.sync_copy(
            rows_v, out_hbm.at[pl.ds(row0 + q * gchunk, gchunk)])


def kernel(class_logits, embedding):
    n, c = class_logits.shape
    _, d = embedding.shape
    rw = n // _NW
    ch = 2 * _L
    nchunk = rw // ch
    mesh = plsc.VectorSubcoreMesh(core_axis_name="c", subcore_axis_name="s")
    body = functools.partial(_sc_body, n, c, d, rw, ch, nchunk)
    f = pl.kernel(
        body,
        out_type=jax.ShapeDtypeStruct((n, d), jnp.float32),
        mesh=mesh,
        compiler_params=pltpu.CompilerParams(needs_layout_passes=False),
        scratch_types=[
            pltpu.VMEM((ch, c), jnp.float32),
            pltpu.VMEM((ch, c), jnp.float32),
            pltpu.VMEM((rw,), jnp.int32),
            pltpu.VMEM((128, d), jnp.float32),
            pltpu.SemaphoreType.DMA,
            pltpu.SemaphoreType.DMA,
            pltpu.SemaphoreType.DMA,
        ],
    )
    return f(class_logits, embedding)


# speedup vs baseline: 2.9250x; 2.9250x over previous
import functools

import jax
import jax.numpy as jnp
from jax import lax
from jax.experimental import pallas as pl
from jax.experimental.pallas import tpu as pltpu
from jax.experimental.pallas import tpu_sc as plsc

_INFO = plsc.get_sparse_core_info()
_NC, _NS, _L = _INFO.num_cores, _INFO.num_subcores, _INFO.num_lanes
_NW = _NC * _NS


def _sc_body(n, c, d, rw, ch, nchunk, x_hbm, emb_hbm, out_hbm,
             x_a, x_b, idx_v, rows_v, sem_a, sem_b, sem_g):
    wid = lax.axis_index("s") * _NC + lax.axis_index("c")
    row0 = wid * rw
    lanes = lax.iota(jnp.int32, _L)
    neg_inf = jnp.full((_L,), -jnp.inf, jnp.float32)
    big = jnp.full((_L,), c, jnp.int32)
    nfull = c // _L
    tail0 = c - _L
    tail_cols = lanes + tail0

    bufs = (x_a, x_b)
    sems = (sem_a, sem_b)

    def start_load(chunk, buf):
        return pltpu.make_async_copy(
            x_hbm.at[pl.ds(row0 + chunk * ch, ch)], bufs[buf], sems[buf])

    def do_chunk(chunk, half):
        buf = bufs[half]
        start_load(chunk, half).wait()
        for g in range(ch // _L):
            acc = jnp.zeros((_L,), jnp.int32)
            for q in range(_L // 4):
                r0 = g * _L + q * 4
                init = (lanes,) + tuple(
                    (neg_inf, jnp.zeros((_L,), jnp.int32))
                    for _ in range(4))

                @plsc.parallel_loop(0, nfull, unroll=2, carry=init)
                def kloop(k, carry, buf=buf, r0=r0):
                    colv, *st = carry
                    out = []
                    for rr in range(4):
                        cm, cc = st[rr]
                        v = buf[r0 + rr, pl.ds(k * _L, _L)]
                        upd = v > cm
                        cm = jnp.where(upd, v, cm)
                        cc = jnp.where(upd, colv, cc)
                        out.append((cm, cc))
                    return (colv + _L,) + tuple(out)

                _, *st = kloop
                for rr in range(4):
                    cm, cc = st[rr]
                    v = buf[r0 + rr, pl.ds(tail0, _L)]
                    upd = v > cm
                    cm = jnp.where(upd, v, cm)
                    cc = jnp.where(upd, tail_cols, cc)
                    m = jnp.max(cm)
                    idx_r = jnp.min(jnp.where(cm == m, cc, big))
                    sel = lanes == (q * 4 + rr)
                    acc = jnp.where(sel, jnp.full((_L,), idx_r, jnp.int32),
                                    acc)
            idx_v[pl.ds(chunk * ch + g * _L, _L)] = acc

    start_load(0, 0).start()
    start_load(1, 1).start()

    def pair_body(cp, carry):
        for half in range(2):
            chunk = cp * 2 + half
            do_chunk(chunk, half)

            @pl.when(chunk + 2 < nchunk)
            def _(chunk=chunk, half=half):
                start_load(chunk + 2, half).start()
        return carry

    lax.fori_loop(0, nchunk // 2, pair_body, 0)

    gchunk = 128
    for q in range(rw // gchunk):
        cp = pltpu.make_async_copy(
            emb_hbm.at[idx_v.at[pl.ds(q * gchunk, gchunk)]], rows_v, sem_g)
        cp.start()
        cp.wait()
        pltpu.sync_copy(
            rows_v, out_hbm.at[pl.ds(row0 + q * gchunk, gchunk)])


def kernel(class_logits, embedding):
    n, c = class_logits.shape
    _, d = embedding.shape
    rw = n // _NW
    ch = 2 * _L
    nchunk = rw // ch
    mesh = plsc.VectorSubcoreMesh(core_axis_name="c", subcore_axis_name="s")
    body = functools.partial(_sc_body, n, c, d, rw, ch, nchunk)
    f = pl.kernel(
        body,
        out_type=jax.ShapeDtypeStruct((n, d), jnp.float32),
        mesh=mesh,
        compiler_params=pltpu.CompilerParams(needs_layout_passes=False),
        scratch_types=[
            pltpu.VMEM((ch, c), jnp.float32),
            pltpu.VMEM((ch, c), jnp.float32),
            pltpu.VMEM((rw,), jnp.int32),
            pltpu.VMEM((128, d), jnp.float32),
            pltpu.SemaphoreType.DMA,
            pltpu.SemaphoreType.DMA,
            pltpu.SemaphoreType.DMA,
        ],
    )
    return f(class_logits, embedding)


# speedup vs baseline: 3.1256x; 1.0686x over previous
import functools

import jax
import jax.numpy as jnp
from jax import lax
from jax.experimental import pallas as pl
from jax.experimental.pallas import tpu as pltpu
from jax.experimental.pallas import tpu_sc as plsc

_INFO = plsc.get_sparse_core_info()
_NC, _NS, _L = _INFO.num_cores, _INFO.num_subcores, _INFO.num_lanes
_NW = _NC * _NS


def _sc_body(row_base, c, rw, ch, nchunk, x_hbm, emb_hbm, out_hbm,
             x_a, x_b, idx_v, rows_v, sem_a, sem_b, sem_g):
    wid = lax.axis_index("s") * _NC + lax.axis_index("c")
    row0 = wid * rw
    lanes = lax.iota(jnp.int32, _L)
    neg_inf = jnp.full((_L,), -jnp.inf, jnp.float32)
    big = jnp.full((_L,), c, jnp.int32)
    nfull = c // _L
    tail0 = c - _L
    tail_cols = lanes + tail0

    bufs = (x_a, x_b)
    sems = (sem_a, sem_b)

    def start_load(chunk, buf):
        return pltpu.make_async_copy(
            x_hbm.at[pl.ds(row_base + row0 + chunk * ch, ch)],
            bufs[buf], sems[buf])

    def do_chunk(chunk, half):
        buf = bufs[half]
        start_load(chunk, half).wait()
        for g in range(ch // _L):
            acc = jnp.zeros((_L,), jnp.int32)
            for q in range(_L // 4):
                r0 = g * _L + q * 4
                init = (lanes,) + tuple(
                    (neg_inf, jnp.zeros((_L,), jnp.int32))
                    for _ in range(4))

                @plsc.parallel_loop(0, nfull, unroll=2, carry=init)
                def kloop(k, carry, buf=buf, r0=r0):
                    colv, *st = carry
                    out = []
                    for rr in range(4):
                        cm, cc = st[rr]
                        v = buf[r0 + rr, pl.ds(k * _L, _L)]
                        upd = v > cm
                        cm = jnp.where(upd, v, cm)
                        cc = jnp.where(upd, colv, cc)
                        out.append((cm, cc))
                    return (colv + _L,) + tuple(out)

                _, *st = kloop
                for rr in range(4):
                    cm, cc = st[rr]
                    v = buf[r0 + rr, pl.ds(tail0, _L)]
                    upd = v > cm
                    cm = jnp.where(upd, v, cm)
                    cc = jnp.where(upd, tail_cols, cc)
                    m = jnp.max(cm)
                    idx_r = jnp.min(jnp.where(cm == m, cc, big))
                    sel = lanes == (q * 4 + rr)
                    acc = jnp.where(sel, jnp.full((_L,), idx_r, jnp.int32),
                                    acc)
            idx_v[pl.ds(chunk * ch + g * _L, _L)] = acc

    start_load(0, 0).start()
    start_load(1, 1).start()

    def pair_body(cp, carry):
        for half in range(2):
            chunk = cp * 2 + half
            do_chunk(chunk, half)

            @pl.when(chunk + 2 < nchunk)
            def _(chunk=chunk, half=half):
                start_load(chunk + 2, half).start()
        return carry

    lax.fori_loop(0, nchunk // 2, pair_body, 0)

    gchunk = 128
    for q in range(rw // gchunk):
        cp = pltpu.make_async_copy(
            emb_hbm.at[idx_v.at[pl.ds(q * gchunk, gchunk)]], rows_v, sem_g)
        cp.start()
        cp.wait()
        pltpu.sync_copy(
            rows_v, out_hbm.at[pl.ds(row0 + q * gchunk, gchunk)])


def _tc_body(x_ref, emb_ref, out_ref):
    x = x_ref[...]
    c = x.shape[1]
    cols = lax.broadcasted_iota(jnp.int32, x.shape, 1)
    xm = jnp.where(cols < c, x, -jnp.inf)
    m = jnp.max(xm, axis=1, keepdims=True)
    idx = jnp.min(jnp.where(xm == m, cols, c), axis=1)
    onehot = (cols == idx[:, None])
    out_ref[...] = jnp.dot(onehot.astype(jnp.float32), emb_ref[...],
                           preferred_element_type=jnp.float32)


def kernel(class_logits, embedding):
    n, c = class_logits.shape
    _, d = embedding.shape
    n_tc = n // 2
    n_sc = n - n_tc
    rw = n_sc // _NW
    ch = 2 * _L
    nchunk = rw // ch

    br = 512
    tc_out = pl.pallas_call(
        _tc_body,
        grid=(n_tc // br,),
        in_specs=[
            pl.BlockSpec((br, c), lambda i: (i, 0)),
            pl.BlockSpec((c, d), lambda i: (0, 0)),
        ],
        out_specs=pl.BlockSpec((br, d), lambda i: (i, 0)),
        out_shape=jax.ShapeDtypeStruct((n_tc, d), jnp.float32),
    )(class_logits, embedding)

    mesh = plsc.VectorSubcoreMesh(core_axis_name="c", subcore_axis_name="s")
    body = functools.partial(_sc_body, n_tc, c, rw, ch, nchunk)
    sc = pl.kernel(
        body,
        out_type=jax.ShapeDtypeStruct((n_sc, d), jnp.float32),
        mesh=mesh,
        compiler_params=pltpu.CompilerParams(needs_layout_passes=False),
        scratch_types=[
            pltpu.VMEM((ch, c), jnp.float32),
            pltpu.VMEM((ch, c), jnp.float32),
            pltpu.VMEM((rw,), jnp.int32),
            pltpu.VMEM((128, d), jnp.float32),
            pltpu.SemaphoreType.DMA,
            pltpu.SemaphoreType.DMA,
            pltpu.SemaphoreType.DMA,
        ],
    )
    sc_out = sc(class_logits, embedding)
    return jnp.concatenate([tc_out, sc_out], axis=0)


# speedup vs baseline: 3.1371x; 1.0037x over previous
import functools

import jax
import jax.numpy as jnp
from jax import lax
from jax.experimental import pallas as pl
from jax.experimental.pallas import tpu as pltpu
from jax.experimental.pallas import tpu_sc as plsc

_INFO = plsc.get_sparse_core_info()
_NC, _NS, _L = _INFO.num_cores, _INFO.num_subcores, _INFO.num_lanes
_NW = _NC * _NS


def _sc_body(row_base, c, rw, ch, nchunk, x_hbm, emb_hbm, out_hbm,
             x_a, x_b, idx_v, rows_v, sem_a, sem_b, sem_g):
    wid = lax.axis_index("s") * _NC + lax.axis_index("c")
    row0 = wid * rw
    lanes = lax.iota(jnp.int32, _L)
    neg_inf = jnp.full((_L,), -jnp.inf, jnp.float32)
    big = jnp.full((_L,), c, jnp.int32)
    nfull = c // _L
    tail0 = c - _L
    tail_cols = lanes + tail0

    bufs = (x_a, x_b)
    sems = (sem_a, sem_b)

    def start_load(chunk, buf):
        return pltpu.make_async_copy(
            x_hbm.at[pl.ds(row_base + row0 + chunk * ch, ch)],
            bufs[buf], sems[buf])

    def do_chunk(chunk, half):
        buf = bufs[half]
        start_load(chunk, half).wait()
        for g in range(ch // _L):
            acc = jnp.zeros((_L,), jnp.int32)
            for q in range(_L // 4):
                r0 = g * _L + q * 4
                init = (lanes,) + tuple(
                    (neg_inf, jnp.zeros((_L,), jnp.int32))
                    for _ in range(4))

                @plsc.parallel_loop(0, nfull, unroll=2, carry=init)
                def kloop(k, carry, buf=buf, r0=r0):
                    colv, *st = carry
                    out = []
                    for rr in range(4):
                        cm, cc = st[rr]
                        v = buf[r0 + rr, pl.ds(k * _L, _L)]
                        upd = v > cm
                        cm = jnp.where(upd, v, cm)
                        cc = jnp.where(upd, colv, cc)
                        out.append((cm, cc))
                    return (colv + _L,) + tuple(out)

                _, *st = kloop
                for rr in range(4):
                    cm, cc = st[rr]
                    v = buf[r0 + rr, pl.ds(tail0, _L)]
                    upd = v > cm
                    cm = jnp.where(upd, v, cm)
                    cc = jnp.where(upd, tail_cols, cc)
                    m = jnp.max(cm)
                    idx_r = jnp.min(jnp.where(cm == m, cc, big))
                    sel = lanes == (q * 4 + rr)
                    acc = jnp.where(sel, jnp.full((_L,), idx_r, jnp.int32),
                                    acc)
            idx_v[pl.ds(chunk * ch + g * _L, _L)] = acc

    start_load(0, 0).start()
    start_load(1, 1).start()

    def pair_body(cp, carry):
        for half in range(2):
            chunk = cp * 2 + half
            do_chunk(chunk, half)

            @pl.when(chunk + 2 < nchunk)
            def _(chunk=chunk, half=half):
                start_load(chunk + 2, half).start()
        return carry

    lax.fori_loop(0, nchunk // 2, pair_body, 0)

    gchunk = 128
    for q in range(rw // gchunk):
        cp = pltpu.make_async_copy(
            emb_hbm.at[idx_v.at[pl.ds(q * gchunk, gchunk)]], rows_v, sem_g)
        cp.start()
        cp.wait()
        pltpu.sync_copy(
            rows_v, out_hbm.at[pl.ds(row0 + q * gchunk, gchunk)])


def _tc_body(x_ref, emb_ref, out_ref):
    x = x_ref[...]
    c = x.shape[1]
    cols = lax.broadcasted_iota(jnp.int32, x.shape, 1)
    xm = jnp.where(cols < c, x, -jnp.inf)
    m = jnp.max(xm, axis=1, keepdims=True)
    idx = jnp.min(jnp.where(xm == m, cols, c), axis=1)
    onehot = (cols == idx[:, None])
    out_ref[...] = jnp.dot(onehot.astype(jnp.float32), emb_ref[...],
                           preferred_element_type=jnp.float32)


def kernel(class_logits, embedding):
    n, c = class_logits.shape
    _, d = embedding.shape
    n_tc = n // 2
    n_sc = n - n_tc
    rw = n_sc // _NW
    ch = 2 * _L
    nchunk = rw // ch

    br = 512
    tc_out = pl.pallas_call(
        _tc_body,
        grid=(n_tc // br,),
        in_specs=[
            pl.BlockSpec((br, c), lambda i: (i, 0)),
            pl.BlockSpec((c, d), lambda i: (0, 0)),
        ],
        out_specs=pl.BlockSpec((br, d), lambda i: (i, 0)),
        out_shape=jax.ShapeDtypeStruct((n_tc, d), jnp.float32),
    )(class_logits, embedding)

    mesh = plsc.VectorSubcoreMesh(core_axis_name="c", subcore_axis_name="s")
    body = functools.partial(_sc_body, n_tc, c, rw, ch, nchunk)
    sc = pl.kernel(
        body,
        out_type=jax.ShapeDtypeStruct((n_sc, d), jnp.float32),
        mesh=mesh,
        compiler_params=pltpu.CompilerParams(needs_layout_passes=False,
                                             use_tc_tiling_on_sc=True),
        scratch_types=[
            pltpu.VMEM((ch, c), jnp.float32),
            pltpu.VMEM((ch, c), jnp.float32),
            pltpu.VMEM((rw,), jnp.int32),
            pltpu.VMEM((128, d), jnp.float32),
            pltpu.SemaphoreType.DMA,
            pltpu.SemaphoreType.DMA,
            pltpu.SemaphoreType.DMA,
        ],
    )
    sc_out = sc(class_logits, embedding)
    return jnp.concatenate([tc_out, sc_out], axis=0)
